# Initial kernel scaffold; baseline (speedup 1.0000x reference)
import jax
import jax.numpy as jnp
from jax.experimental import pallas as pl


def kernel(pitch, gd, vel, P_table, Wgd, bgd, Wv, bv, pos_table):
    raise NotImplementedError("write your pallas kernel here")


# speedup vs baseline: 2.2796x; 2.2796x over previous
import functools

import jax
import jax.numpy as jnp
from jax import lax
from jax.experimental import pallas as pl
from jax.experimental.pallas import tpu as pltpu
from jax.experimental.pallas import tpu_sc as plsc

_NC = 2
_NS = 16
_LANES = 16
_NBUF = 2


def kernel(pitch, gd, vel, P_table, Wgd, bgd, Wv, bv, pos_table):
    B, T = pitch.shape
    NP = P_table.shape[0]
    D_P = P_table.shape[1]
    D_GD = Wgd.shape[1]
    D_V = Wv.shape[1]
    D = D_P + D_GD + D_V

    A = jnp.concatenate(
        [
            pos_table[:, :D_P],
            pos_table[:, D_P:] + bgd[None, :],
            jnp.broadcast_to(bv[None, :], (T, D_V)),
        ],
        axis=1,
    )

    NW = _NC * _NS
    RPW = B // NW

    SW = 4
    s_pack = jnp.concatenate(
        [gd, vel, pitch.astype(jnp.float32)[..., None]], axis=-1
    ).reshape(B, SW * T)
    s_pack = jnp.pad(s_pack, ((0, 0), (0, _LANES)))
    SLEN = SW * T + _LANES

    mesh = plsc.VectorSubcoreMesh(core_axis_name="c", subcore_axis_name="s")

    @functools.partial(
        pl.kernel,
        out_type=jax.ShapeDtypeStruct((B, T, D), jnp.float32),
        mesh=mesh,
        scratch_types=[
            pltpu.VMEM((T, D), jnp.float32),
            pltpu.VMEM((NP, D_P), jnp.float32),
            pltpu.VMEM((2, D_GD), jnp.float32),
            pltpu.VMEM((1, D_V), jnp.float32),
            [pltpu.VMEM((SLEN,), jnp.float32)] * _NBUF,
            pltpu.VMEM((_NBUF, T, D), jnp.float32),
            [pltpu.SemaphoreType.DMA] * _NBUF,
        ],
    )
    def run(s_hbm, p_hbm, a_hbm, wgd_hbm, wv_hbm,
            out_hbm, a_v, p_v, wgd_v, wv_v, s_bufs, out_v, sems):
        wid = lax.axis_index("s") * _NC + lax.axis_index("c")
        pltpu.sync_copy(a_hbm, a_v)
        pltpu.sync_copy(p_hbm, p_v)
        pltpu.sync_copy(wgd_hbm, wgd_v)
        pltpu.sync_copy(wv_hbm, wv_v)

        def compute_row(p):
            def tok_body(t, c):
                srow = s_bufs[p][pl.ds(SW * t, _LANES)]
                gd0 = srow[0]
                gd1 = srow[1]
                vl = srow[2]
                r = srow[3].astype(jnp.int32)
                for q in range(D_P // _LANES):
                    col = _LANES * q
                    out_v[p, t, pl.ds(col, _LANES)] = (
                        p_v[r, pl.ds(col, _LANES)]
                        + a_v[t, pl.ds(col, _LANES)])
                for q in range(D_GD // _LANES):
                    col = D_P + _LANES * q
                    out_v[p, t, pl.ds(col, _LANES)] = (
                        a_v[t, pl.ds(col, _LANES)]
                        + gd0 * wgd_v[0, pl.ds(_LANES * q, _LANES)]
                        + gd1 * wgd_v[1, pl.ds(_LANES * q, _LANES)])
                for q in range(D_V // _LANES):
                    col = D_P + D_GD + _LANES * q
                    out_v[p, t, pl.ds(col, _LANES)] = (
                        a_v[t, pl.ds(col, _LANES)]
                        + vl * wv_v[0, pl.ds(_LANES * q, _LANES)])
                return c

            lax.fori_loop(0, T, tok_body, 0)

        pending = [None] * _NBUF
        for i in range(RPW):
            p = i % _NBUF
            if pending[p] is not None:
                pending[p].wait()
            b = wid * RPW + i
            pltpu.sync_copy(s_hbm.at[b], s_bufs[p])
            compute_row(p)
            pending[p] = pltpu.async_copy(out_v.at[p], out_hbm.at[b], sems[p])
        for p in range(_NBUF):
            if pending[p] is not None:
                pending[p].wait()

    return run(s_pack, P_table, A, Wgd, Wv.reshape(1, D_V))


# speedup vs baseline: 8.2057x; 3.5997x over previous
import functools

import jax
import jax.numpy as jnp
from jax import lax
from jax.experimental import pallas as pl
from jax.experimental.pallas import tpu as pltpu
from jax.experimental.pallas import tpu_sc as plsc

_NC = 2
_NS = 16
_LANES = 16
_NBUF = 2


def kernel(pitch, gd, vel, P_table, Wgd, bgd, Wv, bv, pos_table):
    B, T = pitch.shape
    NP = P_table.shape[0]
    D_P = P_table.shape[1]
    D_GD = Wgd.shape[1]
    D_V = Wv.shape[1]
    D = D_P + D_GD + D_V

    A = jnp.concatenate(
        [
            pos_table[:, :D_P],
            pos_table[:, D_P:] + bgd[None, :],
            jnp.broadcast_to(bv[None, :], (T, D_V)),
        ],
        axis=1,
    )

    NW = _NC * _NS
    RPW = B // NW

    SW = 4
    s_pack = jnp.concatenate(
        [gd, vel, pitch.astype(jnp.float32)[..., None]], axis=-1
    ).reshape(B, SW * T)
    s_pack = jnp.pad(s_pack, ((0, 0), (0, _LANES)))
    SLEN = SW * T + _LANES

    mesh = plsc.VectorSubcoreMesh(core_axis_name="c", subcore_axis_name="s")

    @functools.partial(
        pl.kernel,
        out_type=jax.ShapeDtypeStruct((B, T, D), jnp.float32),
        mesh=mesh,
        scratch_types=[
            pltpu.VMEM((T, D), jnp.float32),
            pltpu.VMEM((NP, D_P), jnp.float32),
            pltpu.VMEM((2, D_GD), jnp.float32),
            pltpu.VMEM((1, D_V), jnp.float32),
            [pltpu.VMEM((SLEN,), jnp.float32)] * _NBUF,
            pltpu.VMEM((_NBUF, T, D), jnp.float32),
            [pltpu.SemaphoreType.DMA] * _NBUF,
        ],
    )
    def run(s_hbm, p_hbm, a_hbm, wgd_hbm, wv_hbm,
            out_hbm, a_v, p_v, wgd_v, wv_v, s_bufs, out_v, sems):
        wid = lax.axis_index("s") * _NC + lax.axis_index("c")
        pltpu.sync_copy(a_hbm, a_v)
        pltpu.sync_copy(p_hbm, p_v)
        pltpu.sync_copy(wgd_hbm, wgd_v)
        pltpu.sync_copy(wv_hbm, wv_v)

        wg = [[wgd_v[k, pl.ds(_LANES * q, _LANES)]
               for q in range(D_GD // _LANES)] for k in range(2)]
        wv = [wv_v[0, pl.ds(_LANES * q, _LANES)]
              for q in range(D_V // _LANES)]

        def compute_row(p):
            @plsc.parallel_loop(0, T, step=1, unroll=4)
            def tok_body(t):
                srow = s_bufs[p][pl.ds(SW * t, _LANES)]
                gd0 = srow[0]
                gd1 = srow[1]
                vl = srow[2]
                r = srow[3].astype(jnp.int32)
                for q in range(D_P // _LANES):
                    col = _LANES * q
                    out_v[p, t, pl.ds(col, _LANES)] = (
                        p_v[r, pl.ds(col, _LANES)]
                        + a_v[t, pl.ds(col, _LANES)])
                for q in range(D_GD // _LANES):
                    col = D_P + _LANES * q
                    out_v[p, t, pl.ds(col, _LANES)] = (
                        a_v[t, pl.ds(col, _LANES)]
                        + gd0 * wg[0][q] + gd1 * wg[1][q])
                for q in range(D_V // _LANES):
                    col = D_P + D_GD + _LANES * q
                    out_v[p, t, pl.ds(col, _LANES)] = (
                        a_v[t, pl.ds(col, _LANES)] + vl * wv[q])

        pending = [None] * _NBUF
        for i in range(RPW):
            p = i % _NBUF
            if pending[p] is not None:
                pending[p].wait()
            b = wid * RPW + i
            pltpu.sync_copy(s_hbm.at[b], s_bufs[p])
            compute_row(p)
            pending[p] = pltpu.async_copy(out_v.at[p], out_hbm.at[b], sems[p])
        for p in range(_NBUF):
            if pending[p] is not None:
                pending[p].wait()

    return run(s_pack, P_table, A, Wgd, Wv.reshape(1, D_V))
